# baseline (device time: 12713 ns/iter reference)
import jax
import jax.numpy as jnp
from jax import lax
from jax.experimental import pallas as pl
from jax.experimental.pallas import tpu as pltpu

N_DEV = 4
R_ORDER = (2, 1, 3, 0)


def kernel(x, w_mat):
    m_per, k = x.shape
    n = w_mat.shape[1]
    n_per = n // N_DEV

    def body(x_ref, w_ref, out_ref, wblk, stage, rbuf,
             copy_sems, send_sems, recv_sems):
        my = lax.axis_index("i")

        barrier_sem = pltpu.get_barrier_semaphore()
        for d in range(N_DEV):
            @pl.when(my != d)
            def _():
                pl.semaphore_signal(
                    barrier_sem, inc=1,
                    device_id=(d,), device_id_type=pl.DeviceIdType.MESH,
                )

        def wdma(slot, r):
            jj = (my + r) % N_DEV
            return pltpu.make_async_copy(
                w_ref.at[:, pl.ds(jj * n_per, n_per)],
                wblk.at[slot],
                copy_sems.at[slot],
            )

        def comm_desc(t, r):
            return pltpu.make_async_remote_copy(
                src_ref=stage.at[t],
                dst_ref=rbuf.at[r],
                send_sem=send_sems.at[t],
                recv_sem=recv_sems.at[r],
                device_id=((my + r) % N_DEV,),
                device_id_type=pl.DeviceIdType.MESH,
            )

        wdma(0, R_ORDER[0]).start()
        xb = x_ref[...].astype(jnp.bfloat16)

        for t, r in enumerate(R_ORDER):
            slot = t % 2
            wdma(slot, r).wait()
            if t + 1 < N_DEV:
                wdma((t + 1) % 2, R_ORDER[t + 1]).start()
            y = jnp.dot(xb, wblk[slot].astype(jnp.bfloat16),
                        preferred_element_type=jnp.float32)
            y = y * jax.nn.sigmoid(y)
            if r == 0:
                out_ref[pl.ds(my * m_per, m_per), :] = y
            else:
                stage[t] = y.astype(jnp.bfloat16)
                if t == 0:
                    pl.semaphore_wait(barrier_sem, N_DEV - 1)
                comm_desc(t, r).start()

        for t, r in ((1, 1), (0, 2), (2, 3)):
            comm_desc(t, r).wait_recv()
            s = (my - r) % N_DEV
            out_ref[pl.ds(s * m_per, m_per), :] = rbuf[r].astype(jnp.float32)

        for t, r in ((0, 2), (1, 1), (2, 3)):
            comm_desc(t, r).wait_send()

    return pl.pallas_call(
        body,
        out_shape=jax.ShapeDtypeStruct((N_DEV * m_per, n_per), jnp.float32),
        in_specs=[
            pl.BlockSpec(memory_space=pltpu.VMEM),
            pl.BlockSpec(memory_space=pltpu.VMEM),
        ],
        out_specs=pl.BlockSpec(memory_space=pltpu.VMEM),
        scratch_shapes=[
            pltpu.VMEM((2, k, n_per), jnp.float32),
            pltpu.VMEM((N_DEV - 1, m_per, n_per), jnp.bfloat16),
            pltpu.VMEM((N_DEV, m_per, n_per), jnp.bfloat16),
            pltpu.SemaphoreType.DMA((2,)),
            pltpu.SemaphoreType.DMA((N_DEV - 1,)),
            pltpu.SemaphoreType.DMA((N_DEV,)),
        ],
        compiler_params=pltpu.CompilerParams(collective_id=0),
    )(x, w_mat)


# device time: 12316 ns/iter; 1.0322x vs baseline; 1.0322x over previous
import jax
import jax.numpy as jnp
from jax import lax
from jax.experimental import pallas as pl
from jax.experimental.pallas import tpu as pltpu

N_DEV = 4
R_ORDER = (2, 1, 3, 0)


def kernel(x, w_mat):
    m_per, k = x.shape
    n = w_mat.shape[1]
    n_per = n // N_DEV

    def body(x_ref, w_ref, out_ref, stage, rbuf, send_sems, recv_sems):
        my = lax.axis_index("i")

        barrier_sem = pltpu.get_barrier_semaphore()
        for d in range(N_DEV):
            @pl.when(my != d)
            def _():
                pl.semaphore_signal(
                    barrier_sem, inc=1,
                    device_id=(d,), device_id_type=pl.DeviceIdType.MESH,
                )

        def schedule(v):
            def comm_desc(t, r):
                return pltpu.make_async_remote_copy(
                    src_ref=stage.at[t],
                    dst_ref=rbuf.at[r],
                    send_sem=send_sems.at[t],
                    recv_sem=recv_sems.at[r],
                    device_id=((v + r) % N_DEV,),
                    device_id_type=pl.DeviceIdType.MESH,
                )

            xb = x_ref[...].astype(jnp.bfloat16)
            for t, r in enumerate(R_ORDER):
                j = (v + r) % N_DEV
                y = jnp.dot(
                    xb, w_ref[:, j * n_per:(j + 1) * n_per].astype(jnp.bfloat16),
                    preferred_element_type=jnp.float32)
                y = y * jax.nn.sigmoid(y)
                if r == 0:
                    out_ref[j * m_per:(j + 1) * m_per, :] = y
                else:
                    stage[t] = y.astype(jnp.bfloat16)
                    if t == 0:
                        pl.semaphore_wait(barrier_sem, N_DEV - 1)
                    comm_desc(t, r).start()

            for t, r in ((1, 1), (0, 2), (2, 3)):
                comm_desc(t, r).wait_recv()
                s = (v - r) % N_DEV
                out_ref[s * m_per:(s + 1) * m_per, :] = rbuf[r].astype(jnp.float32)

            for t, r in ((0, 2), (1, 1), (2, 3)):
                comm_desc(t, r).wait_send()

        for v in range(N_DEV):
            @pl.when(my == v)
            def _(v=v):
                schedule(v)

    return pl.pallas_call(
        body,
        out_shape=jax.ShapeDtypeStruct((N_DEV * m_per, n_per), jnp.float32),
        in_specs=[
            pl.BlockSpec(memory_space=pltpu.VMEM),
            pl.BlockSpec(memory_space=pltpu.VMEM),
        ],
        out_specs=pl.BlockSpec(memory_space=pltpu.VMEM),
        scratch_shapes=[
            pltpu.VMEM((N_DEV - 1, m_per, n_per), jnp.bfloat16),
            pltpu.VMEM((N_DEV, m_per, n_per), jnp.bfloat16),
            pltpu.SemaphoreType.DMA((N_DEV - 1,)),
            pltpu.SemaphoreType.DMA((N_DEV,)),
        ],
        compiler_params=pltpu.CompilerParams(collective_id=0),
    )(x, w_mat)
